# baseline (device time: 81436 ns/iter reference)
import jax
import jax.numpy as jnp
from jax import lax
from jax.experimental import pallas as pl
from jax.experimental.pallas import tpu as pltpu

N_DEV = 16
N_TOK = 256
D_IN = 128
H_OUT = 256
N_EXP = 32
CAP = 6


def kernel(x, router_W, route_idx, expert_W):
    del router_W

    def body(x_ref, route_ref, ew_ref, out_ref, comm_ref, send_sems, recv_sems):
        my_pos = lax.axis_index("i")
        right = lax.rem(my_pos + 1, N_DEV)

        route = route_ref[:, :]
        onehot = (
            route == lax.broadcasted_iota(jnp.int32, (N_TOK, N_EXP), 1)
        ).astype(jnp.float32)
        tri = (
            lax.broadcasted_iota(jnp.int32, (N_TOK, N_TOK), 0)
            >= lax.broadcasted_iota(jnp.int32, (N_TOK, N_TOK), 1)
        ).astype(jnp.float32)
        cum = jnp.dot(tri, onehot, preferred_element_type=jnp.float32)
        rank = jnp.sum(onehot * cum, axis=1, keepdims=True)
        under_cap = rank <= float(CAP)

        e0 = my_pos * 2
        is0 = jnp.logical_and(route == e0, under_cap)
        is1 = jnp.logical_and(route == e0 + 1, under_cap)
        xv = x_ref[:, :]
        y0 = jnp.dot(xv, ew_ref[0], preferred_element_type=jnp.float32)
        y1 = jnp.dot(xv, ew_ref[1], preferred_element_type=jnp.float32)
        partial = jnp.where(is0, y0, 0.0) + jnp.where(is1, y1, 0.0)

        comm_ref[0, :, :] = partial
        acc = partial
        for h in range(N_DEV - 1):
            rdma = pltpu.make_async_remote_copy(
                src_ref=comm_ref.at[h],
                dst_ref=comm_ref.at[h + 1],
                send_sem=send_sems.at[h],
                recv_sem=recv_sems.at[h],
                device_id=(right,),
                device_id_type=pl.DeviceIdType.MESH,
            )
            rdma.start()
            rdma.wait()
            acc = acc + comm_ref[h + 1, :, :]
        out_ref[:, :] = acc

    return pl.pallas_call(
        body,
        out_shape=jax.ShapeDtypeStruct((N_TOK, H_OUT), jnp.float32),
        in_specs=[
            pl.BlockSpec(memory_space=pltpu.VMEM),
            pl.BlockSpec(memory_space=pltpu.VMEM),
            pl.BlockSpec(memory_space=pltpu.VMEM),
        ],
        out_specs=pl.BlockSpec(memory_space=pltpu.VMEM),
        scratch_shapes=[
            pltpu.VMEM((N_DEV, N_TOK, H_OUT), jnp.float32),
            pltpu.SemaphoreType.DMA((N_DEV - 1,)),
            pltpu.SemaphoreType.DMA((N_DEV - 1,)),
        ],
    )(x, route_idx, expert_W)


# device time: 34527 ns/iter; 2.3586x vs baseline; 2.3586x over previous
import jax
import jax.numpy as jnp
from jax import lax
from jax.experimental import pallas as pl
from jax.experimental.pallas import tpu as pltpu

N_DEV = 16
N_TOK = 256
D_IN = 128
H_OUT = 256
N_EXP = 32
CAP = 6


def kernel(x, router_W, route_idx, expert_W):
    del router_W

    n_stage = N_DEV.bit_length() - 1

    def body(x_ref, route_ref, ew_ref, out_ref, send_ref, recv_ref,
             send_sems, recv_sems):
        my_pos = lax.axis_index("i")

        route = route_ref[:, :]
        onehot = (
            route == lax.broadcasted_iota(jnp.int32, (N_TOK, N_EXP), 1)
        ).astype(jnp.float32)
        tri = (
            lax.broadcasted_iota(jnp.int32, (N_TOK, N_TOK), 0)
            >= lax.broadcasted_iota(jnp.int32, (N_TOK, N_TOK), 1)
        ).astype(jnp.float32)
        cum = jnp.dot(tri, onehot, preferred_element_type=jnp.float32)
        rank = jnp.sum(onehot * cum, axis=1, keepdims=True)
        under_cap = rank <= float(CAP)

        e0 = my_pos * 2
        is0 = jnp.logical_and(route == e0, under_cap)
        is1 = jnp.logical_and(route == e0 + 1, under_cap)
        xv = x_ref[:, :]
        y0 = jnp.dot(xv, ew_ref[0], preferred_element_type=jnp.float32)
        y1 = jnp.dot(xv, ew_ref[1], preferred_element_type=jnp.float32)
        partial = jnp.where(is0, y0, 0.0) + jnp.where(is1, y1, 0.0)

        acc = partial
        for k in range(n_stage):
            partner = jnp.bitwise_xor(my_pos, 1 << k)
            send_ref[:, :] = acc
            rdma = pltpu.make_async_remote_copy(
                src_ref=send_ref,
                dst_ref=recv_ref.at[k],
                send_sem=send_sems.at[k],
                recv_sem=recv_sems.at[k],
                device_id=(partner,),
                device_id_type=pl.DeviceIdType.MESH,
            )
            rdma.start()
            rdma.wait()
            acc = acc + recv_ref[k, :, :]
        out_ref[:, :] = acc

    return pl.pallas_call(
        body,
        out_shape=jax.ShapeDtypeStruct((N_TOK, H_OUT), jnp.float32),
        in_specs=[
            pl.BlockSpec(memory_space=pltpu.VMEM),
            pl.BlockSpec(memory_space=pltpu.VMEM),
            pl.BlockSpec(memory_space=pltpu.VMEM),
        ],
        out_specs=pl.BlockSpec(memory_space=pltpu.VMEM),
        scratch_shapes=[
            pltpu.VMEM((N_TOK, H_OUT), jnp.float32),
            pltpu.VMEM((n_stage, N_TOK, H_OUT), jnp.float32),
            pltpu.SemaphoreType.DMA((n_stage,)),
            pltpu.SemaphoreType.DMA((n_stage,)),
        ],
    )(x, route_idx, expert_W)


# device time: 17782 ns/iter; 4.5797x vs baseline; 1.9417x over previous
import jax
import jax.numpy as jnp
from jax import lax
from jax.experimental import pallas as pl
from jax.experimental.pallas import tpu as pltpu

N_DEV = 16
N_TOK = 256
D_IN = 128
H_OUT = 256
N_EXP = 32
CAP = 6
BLOB = 16


def kernel(x, router_W, route_idx, expert_W):
    del router_W

    def body(x_ref, route_ref, ew_ref, out_ref, gather_ref,
             send_sems, recv_sems):
        my_pos = lax.axis_index("i")
        off = my_pos * BLOB

        route = route_ref[:, :]
        onehot = (
            route == lax.broadcasted_iota(jnp.int32, (N_TOK, N_EXP), 1)
        ).astype(jnp.float32)
        tri = (
            lax.broadcasted_iota(jnp.int32, (N_TOK, N_TOK), 0)
            >= lax.broadcasted_iota(jnp.int32, (N_TOK, N_TOK), 1)
        ).astype(jnp.float32)
        cum = jnp.dot(tri, onehot, preferred_element_type=jnp.float32)
        rank = jnp.sum(onehot * cum, axis=1, keepdims=True).astype(
            jnp.int32
        )
        under_cap = rank <= CAP

        e0 = my_pos * 2
        mine = jnp.logical_and(
            jnp.logical_and(route >= e0, route <= e0 + 1), under_cap
        )
        myslot = jnp.where(mine, (route - e0) * CAP + rank - 1, -1)
        T = (
            myslot == lax.broadcasted_iota(jnp.int32, (N_TOK, BLOB), 1)
        ).astype(jnp.float32)
        xv = x_ref[:, :]
        y0 = jnp.dot(xv, ew_ref[0], preferred_element_type=jnp.float32)
        y1 = jnp.dot(xv, ew_ref[1], preferred_element_type=jnp.float32)
        ysel = jnp.where(route == e0, y0, y1)
        blob = lax.dot_general(
            T, ysel, (((0,), (0,)), ((), ())),
            preferred_element_type=jnp.float32,
        )
        gather_ref[pl.ds(off, BLOB), :] = blob

        sends = []
        for k in range(1, N_DEV):
            tgt = lax.rem(my_pos + k, N_DEV)
            rdma = pltpu.make_async_remote_copy(
                src_ref=gather_ref.at[pl.ds(off, BLOB)],
                dst_ref=gather_ref.at[pl.ds(off, BLOB)],
                send_sem=send_sems.at[tgt],
                recv_sem=recv_sems.at[my_pos],
                device_id=(tgt,),
                device_id_type=pl.DeviceIdType.MESH,
            )
            rdma.start()
            sends.append(rdma)
        for k in range(1, N_DEV):
            src_dev = lax.rem(my_pos + N_DEV - k, N_DEV)
            recv = pltpu.make_async_remote_copy(
                src_ref=gather_ref.at[pl.ds(off, BLOB)],
                dst_ref=gather_ref.at[pl.ds(src_dev * BLOB, BLOB)],
                send_sem=send_sems.at[src_dev],
                recv_sem=recv_sems.at[src_dev],
                device_id=(src_dev,),
                device_id_type=pl.DeviceIdType.MESH,
            )
            recv.wait_recv()
        for rdma in sends:
            rdma.wait_send()

        d_of = route // 2
        le_of = route - d_of * 2
        col = jnp.where(
            under_cap, d_of * BLOB + le_of * CAP + rank - 1, -1
        )
        P = (
            col == lax.broadcasted_iota(jnp.int32, (N_TOK, N_DEV * BLOB), 1)
        ).astype(jnp.float32)
        out_ref[:, :] = jnp.dot(
            P, gather_ref[:, :], preferred_element_type=jnp.float32
        )

    return pl.pallas_call(
        body,
        out_shape=jax.ShapeDtypeStruct((N_TOK, H_OUT), jnp.float32),
        in_specs=[
            pl.BlockSpec(memory_space=pltpu.VMEM),
            pl.BlockSpec(memory_space=pltpu.VMEM),
            pl.BlockSpec(memory_space=pltpu.VMEM),
        ],
        out_specs=pl.BlockSpec(memory_space=pltpu.VMEM),
        scratch_shapes=[
            pltpu.VMEM((N_DEV * BLOB, H_OUT), jnp.float32),
            pltpu.SemaphoreType.DMA((N_DEV,)),
            pltpu.SemaphoreType.DMA((N_DEV,)),
        ],
    )(x, route_idx, expert_W)


# device time: 12362 ns/iter; 6.5876x vs baseline; 1.4384x over previous
import jax
import jax.numpy as jnp
from jax import lax
from jax.experimental import pallas as pl
from jax.experimental.pallas import tpu as pltpu

N_DEV = 16
N_TOK = 256
D_IN = 128
H_OUT = 256
N_EXP = 32
CAP = 6
BLOB = 16


def kernel(x, router_W, route_idx, expert_W):
    del router_W

    def body(x_ref, route_ref, ew_ref, out_ref, gather_ref,
             send_sems, recv_sems):
        my_pos = lax.axis_index("i")
        off = my_pos * BLOB

        barrier_sem = pltpu.get_barrier_semaphore()
        for k in range(1, N_DEV):
            pl.semaphore_signal(
                barrier_sem, inc=1,
                device_id=(lax.rem(my_pos + k, N_DEV),),
                device_id_type=pl.DeviceIdType.MESH,
            )

        route = route_ref[:, :]
        onehot = (
            route == lax.broadcasted_iota(jnp.int32, (N_TOK, N_EXP), 1)
        ).astype(jnp.float32)
        tri = (
            lax.broadcasted_iota(jnp.int32, (N_TOK, N_TOK), 0)
            >= lax.broadcasted_iota(jnp.int32, (N_TOK, N_TOK), 1)
        ).astype(jnp.float32)
        cum = jnp.dot(tri, onehot, preferred_element_type=jnp.float32)
        rank = jnp.sum(onehot * cum, axis=1, keepdims=True).astype(
            jnp.int32
        )
        under_cap = rank <= CAP

        e0 = my_pos * 2
        mine = jnp.logical_and(
            jnp.logical_and(route >= e0, route <= e0 + 1), under_cap
        )
        myslot = jnp.where(mine, (route - e0) * CAP + rank - 1, -1)
        T = (
            myslot == lax.broadcasted_iota(jnp.int32, (N_TOK, BLOB), 1)
        ).astype(jnp.float32)
        gx = lax.dot_general(
            T, x_ref[:, :], (((0,), (0,)), ((), ())),
            preferred_element_type=jnp.float32,
        )
        b0 = jnp.dot(gx, ew_ref[0], preferred_element_type=jnp.float32)
        b1 = jnp.dot(gx, ew_ref[1], preferred_element_type=jnp.float32)
        slot_row = lax.broadcasted_iota(jnp.int32, (BLOB, 1), 0)
        blob = jnp.where(slot_row < CAP, b0, b1)
        gather_ref[pl.ds(off, BLOB), :] = blob

        pl.semaphore_wait(barrier_sem, N_DEV - 1)
        sends = []
        for k in range(1, N_DEV):
            tgt = lax.rem(my_pos + k, N_DEV)
            rdma = pltpu.make_async_remote_copy(
                src_ref=gather_ref.at[pl.ds(off, BLOB)],
                dst_ref=gather_ref.at[pl.ds(off, BLOB)],
                send_sem=send_sems.at[tgt],
                recv_sem=recv_sems.at[my_pos],
                device_id=(tgt,),
                device_id_type=pl.DeviceIdType.MESH,
            )
            rdma.start()
            sends.append(rdma)
        d_of = route // 2
        le_of = route - d_of * 2
        col = jnp.where(
            under_cap, d_of * BLOB + le_of * CAP + rank - 1, -1
        )
        P = (
            col == lax.broadcasted_iota(jnp.int32, (N_TOK, N_DEV * BLOB), 1)
        ).astype(jnp.float32)

        for k in range(1, N_DEV):
            src_dev = lax.rem(my_pos + N_DEV - k, N_DEV)
            recv = pltpu.make_async_remote_copy(
                src_ref=gather_ref.at[pl.ds(off, BLOB)],
                dst_ref=gather_ref.at[pl.ds(src_dev * BLOB, BLOB)],
                send_sem=send_sems.at[src_dev],
                recv_sem=recv_sems.at[src_dev],
                device_id=(src_dev,),
                device_id_type=pl.DeviceIdType.MESH,
            )
            recv.wait_recv()

        out_ref[:, :] = jnp.dot(
            P, gather_ref[:, :], preferred_element_type=jnp.float32
        )
        for rdma in sends:
            rdma.wait_send()

    return pl.pallas_call(
        body,
        out_shape=jax.ShapeDtypeStruct((N_TOK, H_OUT), jnp.float32),
        in_specs=[
            pl.BlockSpec(memory_space=pltpu.VMEM),
            pl.BlockSpec(memory_space=pltpu.VMEM),
            pl.BlockSpec(memory_space=pltpu.VMEM),
        ],
        out_specs=pl.BlockSpec(memory_space=pltpu.VMEM),
        scratch_shapes=[
            pltpu.VMEM((N_DEV * BLOB, H_OUT), jnp.float32),
            pltpu.SemaphoreType.DMA((N_DEV,)),
            pltpu.SemaphoreType.DMA((N_DEV,)),
        ],
        compiler_params=pltpu.CompilerParams(collective_id=0),
    )(x, route_idx, expert_W)
